# baseline (speedup 1.0000x reference)
import functools

import jax
import jax.numpy as jnp
from jax import lax
from jax.experimental import pallas as pl
from jax.experimental.pallas import tpu as pltpu
from jax.experimental.pallas import tpu_sc as plsc

B = 50000
D = 3072
N = 1024
NC, NS, L = 2, 16, 16
NW = NC * NS
R = 1568
LAST_R = B - (NW - 1) * R
NCHUNK = N // L
CAP = N + L
G = 8
_MESH = plsc.VectorSubcoreMesh(core_axis_name="c", subcore_axis_name="s")
_PARAMS = pltpu.CompilerParams(needs_layout_passes=False)


def _route_body(memy, memt, yin, tin, inds, outy, outt, jl_out, dl2_out,
                cnt_out, inds_v, table, s16, jlist, dlist, dlist2, cntbuf,
                yall, tall, yrange, trange):
    sid = lax.axis_index("s")
    wid = sid * NC + lax.axis_index("c")
    lo = wid * R
    is_last = wid == NW - 1
    iota = lax.iota(jnp.int32, L)

    pltpu.sync_copy(inds, inds_v)
    pltpu.sync_copy(yin, yall)
    pltpu.sync_copy(tin, tall)

    @pl.when(jnp.logical_not(is_last))
    def _():
        pltpu.sync_copy(memy.at[pl.ds(lo, R)], yrange.at[pl.ds(0, R)])
        pltpu.sync_copy(memt.at[pl.ds(lo, R)], trange.at[pl.ds(0, R)])

    @pl.when(is_last)
    def _():
        pltpu.sync_copy(memy.at[pl.ds(lo, LAST_R)], yrange.at[pl.ds(0, LAST_R)])
        pltpu.sync_copy(memt.at[pl.ds(lo, LAST_R)], trange.at[pl.ds(0, LAST_R)])

    def init_tab(i, _):
        table[pl.ds(i * L, L)] = jnp.full((L,), -1, jnp.int32)
        return 0
    lax.fori_loop(0, R // L, init_tab, 0)

    def pass_a(c, _):
        iv = inds_v[pl.ds(c * L, L)]
        jv = iota + c * L
        s16[...] = iv
        dup = jnp.zeros((L,), jnp.int32)
        for s in range(1, L):
            nxt = plsc.load_gather(s16, [jnp.minimum(iota + s, L - 1)])
            valid = (iota + s) <= (L - 1)
            dup = jnp.where(jnp.logical_and(valid, nxt == iv), 1, dup)
        winlane = dup == 0
        local = iv - lo
        inr = jnp.logical_and(local >= 0, local < R)
        localc = jnp.clip(local, 0, R - 1)
        plsc.store_scatter(table, [localc], jv,
                           mask=jnp.logical_and(winlane, inr))
        return 0
    lax.fori_loop(0, NCHUNK, pass_a, 0)

    def pass_b(c, cnt):
        iv = inds_v[pl.ds(c * L, L)]
        jv = iota + c * L
        local = iv - lo
        inr = jnp.logical_and(local >= 0, local < R)
        localc = jnp.clip(local, 0, R - 1)
        tv = plsc.load_gather(table, [localc])
        win = jnp.logical_and(inr, tv == jv)
        wc = plsc.cumsum(win.astype(jnp.int32))
        pos = jnp.clip(cnt + wc - 1, 0, CAP - 1)
        plsc.store_scatter(jlist, [pos], jv, mask=win)
        plsc.store_scatter(dlist, [pos], iv, mask=win)
        plsc.store_scatter(dlist2, [pos // G, pos - (pos // G) * G], iv,
                           mask=win)
        return cnt + jnp.max(wc)
    cnt = lax.fori_loop(0, NCHUNK, pass_b, jnp.int32(0))

    lastp = jnp.full((L,), jnp.clip(cnt - 1, 0, CAP - 1), jnp.int32)
    jlast = plsc.load_gather(jlist, [lastp])
    dlast = plsc.load_gather(dlist, [lastp])
    padp = jnp.clip(cnt + iota, 0, CAP - 1)
    plsc.store_scatter(jlist, [padp], jlast)
    plsc.store_scatter(dlist, [padp], dlast)
    plsc.store_scatter(dlist2, [padp // G, padp - (padp // G) * G], dlast)

    cntbuf[...] = jnp.full((L,), cnt, jnp.int32)
    pltpu.sync_copy(jlist, jl_out.at[wid])
    pltpu.sync_copy(dlist2, dl2_out.at[wid])
    pltpu.sync_copy(cntbuf, cnt_out.at[wid])

    nch16 = (cnt + L - 1) // L

    def yt_chunk(i, _):
        jv = jlist[pl.ds(i * L, L)]
        dv = dlist[pl.ds(i * L, L)]
        ldv = dv - lo
        plsc.store_scatter(yrange, [ldv], plsc.load_gather(yall, [jv]))
        plsc.store_scatter(trange, [ldv], plsc.load_gather(tall, [jv]))
        return 0
    lax.fori_loop(0, nch16, yt_chunk, 0)

    @pl.when(jnp.logical_not(is_last))
    def _():
        pltpu.sync_copy(yrange.at[pl.ds(0, R)], outy.at[pl.ds(lo, R)])
        pltpu.sync_copy(trange.at[pl.ds(0, R)], outt.at[pl.ds(lo, R)])

    @pl.when(is_last)
    def _():
        pltpu.sync_copy(yrange.at[pl.ds(0, LAST_R)], outy.at[pl.ds(lo, LAST_R)])
        pltpu.sync_copy(trange.at[pl.ds(0, LAST_R)], outt.at[pl.ds(lo, LAST_R)])


_route_call = functools.partial(
    pl.kernel,
    out_type=(
        jax.ShapeDtypeStruct((B,), jnp.float32),
        jax.ShapeDtypeStruct((B,), jnp.int32),
        jax.ShapeDtypeStruct((NW, CAP), jnp.int32),
        jax.ShapeDtypeStruct((NW, CAP // G, G), jnp.int32),
        jax.ShapeDtypeStruct((NW, L), jnp.int32),
    ),
    mesh=_MESH,
    compiler_params=_PARAMS,
    scratch_types=[
        pltpu.VMEM((N,), jnp.int32),
        pltpu.VMEM((R,), jnp.int32),
        pltpu.VMEM((L,), jnp.int32),
        pltpu.VMEM((CAP,), jnp.int32),
        pltpu.VMEM((CAP,), jnp.int32),
        pltpu.VMEM((CAP // G, G), jnp.int32),
        pltpu.VMEM((L,), jnp.int32),
        pltpu.VMEM((N,), jnp.float32),
        pltpu.VMEM((N,), jnp.int32),
        pltpu.VMEM((R,), jnp.float32),
        pltpu.VMEM((R,), jnp.int32),
    ],
)(_route_body)


def _scatter_body(Xin, jl_in, dl2_in, cnt_in, outX,
                  jlist, dlist2, cntbuf, ubuf0, ubuf1,
                  gsem0, gsem1, ssem0, ssem1):
    sid = lax.axis_index("s")
    wid = sid * NC + lax.axis_index("c")

    pltpu.sync_copy(jl_in.at[wid], jlist)
    pltpu.sync_copy(dl2_in.at[wid], dlist2)
    pltpu.sync_copy(cnt_in.at[wid], cntbuf)
    cnt = jnp.max(cntbuf[...])

    nchu = (cnt + G - 1) // G

    @pl.when(nchu > 0)
    def _():
        pltpu.async_copy(Xin.at[jlist.at[pl.ds(0, G)]], ubuf0, gsem0)

    @pl.when(nchu > 1)
    def _():
        pltpu.async_copy(Xin.at[jlist.at[pl.ds(G, G)]], ubuf1, gsem1)

    def update_chunk(i, _):
        def turn(buf, gsem, ssem):
            pltpu.make_async_copy(Xin.at[jlist.at[pl.ds(i * G, G)]],
                                  buf, gsem).wait()
            pltpu.async_copy(buf, outX.at[dlist2.at[i]], ssem)
            pltpu.make_async_copy(buf, outX.at[dlist2.at[i]], ssem).wait()

            @pl.when(i + 2 < nchu)
            def _():
                pltpu.async_copy(Xin.at[jlist.at[pl.ds((i + 2) * G, G)]],
                                 buf, gsem)

        @pl.when(i % 2 == 0)
        def _():
            turn(ubuf0, gsem0, ssem0)

        @pl.when(i % 2 == 1)
        def _():
            turn(ubuf1, gsem1, ssem1)

        return 0
    lax.fori_loop(0, nchu, update_chunk, 0)


_scatter_call = functools.partial(
    pl.kernel,
    out_type=(),
    mesh=_MESH,
    compiler_params=_PARAMS,
    scratch_types=[
        pltpu.VMEM((CAP,), jnp.int32),
        pltpu.VMEM((CAP // G, G), jnp.int32),
        pltpu.VMEM((L,), jnp.int32),
        pltpu.VMEM((G, D), jnp.float32),
        pltpu.VMEM((G, D), jnp.float32),
        pltpu.SemaphoreType.DMA,
        pltpu.SemaphoreType.DMA,
        pltpu.SemaphoreType.DMA,
        pltpu.SemaphoreType.DMA,
    ],
)(_scatter_body)


def kernel(mem_X, mem_y, mem_task_ids, X, y, task_ids, inds):
    out_y, out_t, jl, dl2, cnts = _route_call(
        mem_y, mem_task_ids, y, task_ids.astype(jnp.int32),
        inds.astype(jnp.int32))
    xref = jax.new_ref(mem_X.reshape(B, D))
    _scatter_call(X.reshape(N, D), jl, dl2, cnts, xref)
    out_X = jax.freeze(xref)
    return (out_X.reshape(mem_X.shape), out_y, out_t)
